# Initial kernel scaffold; baseline (speedup 1.0000x reference)
import jax
import jax.numpy as jnp
from jax.experimental import pallas as pl


def kernel(x, PE_weight):
    raise NotImplementedError("write your pallas kernel here")


# speedup vs baseline: 1.9862x; 1.9862x over previous
import functools

import jax
import jax.numpy as jnp
from jax import lax
from jax.experimental import pallas as pl
from jax.experimental.pallas import tpu as pltpu
from jax.experimental.pallas import tpu_sc as plsc

CTX_LENGTH = 8192
EMBED_SIZE = 1024
BATCH = 4
SEQ = 8192

N_ROWS = BATCH * SEQ
NUM_CORES = 2
NUM_SUBCORES = 16
NW = NUM_CORES * NUM_SUBCORES
ROWS_PER_W = N_ROWS // NW
CHUNK = 32
NBUF = 2
NCHUNKS = ROWS_PER_W // CHUNK


@functools.partial(
    pl.kernel,
    mesh=plsc.VectorSubcoreMesh(core_axis_name="c", subcore_axis_name="s"),
    out_type=jax.ShapeDtypeStruct((N_ROWS, EMBED_SIZE), jnp.float32),
    scratch_types=[
        pltpu.VMEM((NCHUNKS, CHUNK), jnp.int32),
        pltpu.VMEM((NBUF, CHUNK, EMBED_SIZE), jnp.float32),
        pltpu.SemaphoreType.DMA,
    ],
)
def _gather_rows(idx_hbm, table_hbm, out_hbm, idx_v, rows_v, gsem):
    wid = lax.axis_index("s") * NUM_CORES + lax.axis_index("c")
    base = wid * ROWS_PER_W
    pltpu.sync_copy(idx_hbm.at[wid], idx_v)

    def body(i, carry):
        for b in range(NBUF):
            j = i * NBUF + b
            pltpu.async_copy(table_hbm.at[idx_v.at[j]], rows_v.at[b], gsem).wait()
            pltpu.sync_copy(rows_v.at[b],
                            out_hbm.at[pl.ds(base + j * CHUNK, CHUNK)])
        return carry

    lax.fori_loop(0, NCHUNKS // NBUF, body, 0)


def kernel(x, PE_weight):
    idx = x.reshape(NW, NCHUNKS, CHUNK).astype(jnp.int32)
    out = _gather_rows(idx, PE_weight)
    return out.reshape(BATCH, SEQ, EMBED_SIZE)


# speedup vs baseline: 2.2444x; 1.1300x over previous
import functools

import jax
import jax.numpy as jnp
from jax import lax
from jax.experimental import pallas as pl
from jax.experimental.pallas import tpu as pltpu
from jax.experimental.pallas import tpu_sc as plsc

CTX_LENGTH = 8192
EMBED_SIZE = 1024
BATCH = 4
SEQ = 8192

N_ROWS = BATCH * SEQ
NUM_CORES = 2
NUM_SUBCORES = 16
NW = NUM_CORES * NUM_SUBCORES
ROWS_PER_W = N_ROWS // NW
CHUNK = 32
NBUF = 2
NCHUNKS = ROWS_PER_W // CHUNK


@functools.partial(
    pl.kernel,
    mesh=plsc.VectorSubcoreMesh(core_axis_name="c", subcore_axis_name="s"),
    out_type=jax.ShapeDtypeStruct((N_ROWS, EMBED_SIZE), jnp.float32),
    scratch_types=[
        pltpu.VMEM((NCHUNKS, CHUNK), jnp.int32),
        pltpu.VMEM((NBUF, CHUNK, EMBED_SIZE), jnp.float32),
    ]
    + [pltpu.SemaphoreType.DMA] * (2 * NBUF),
)
def _gather_rows(idx_hbm, table_hbm, out_hbm, idx_v, rows_v, *sems):
    gsem, osem = sems[:NBUF], sems[NBUF:]
    wid = lax.axis_index("s") * NUM_CORES + lax.axis_index("c")
    base = wid * ROWS_PER_W
    pltpu.sync_copy(idx_hbm.at[wid], idx_v)

    def start_gather(j, b):
        pltpu.async_copy(table_hbm.at[idx_v.at[j]], rows_v.at[b], gsem[b])

    def start_out(j, b):
        pltpu.async_copy(rows_v.at[b],
                         out_hbm.at[pl.ds(base + j * CHUNK, CHUNK)], osem[b])

    def wait(sem):
        pltpu.make_async_copy(
            rows_v.at[0], out_hbm.at[pl.ds(base, CHUNK)], sem).wait()

    for b in range(NBUF):
        start_gather(b, b)
    for b in range(NBUF):
        wait(gsem[b])
        start_out(b, b)

    def body(i, carry):
        for b in range(NBUF):
            wait(osem[b])
            start_gather(i * NBUF + b, b)
        for b in range(NBUF):
            wait(gsem[b])
            start_out(i * NBUF + b, b)
        return carry

    lax.fori_loop(1, NCHUNKS // NBUF, body, 0)
    for b in range(NBUF):
        wait(osem[b])


def kernel(x, PE_weight):
    idx = x.reshape(NW, NCHUNKS, CHUNK).astype(jnp.int32)
    out = _gather_rows(idx, PE_weight)
    return out.reshape(BATCH, SEQ, EMBED_SIZE)


# speedup vs baseline: 2.3028x; 1.0260x over previous
import functools

import jax
import jax.numpy as jnp
from jax import lax
from jax.experimental import pallas as pl
from jax.experimental.pallas import tpu as pltpu
from jax.experimental.pallas import tpu_sc as plsc

CTX_LENGTH = 8192
EMBED_SIZE = 1024
BATCH = 4
SEQ = 8192

N_ROWS = BATCH * SEQ
NUM_CORES = 2
NUM_SUBCORES = 16
NW = NUM_CORES * NUM_SUBCORES
ROWS_PER_W = N_ROWS // NW
CHUNK = 16
NBUF = 4
NCHUNKS = ROWS_PER_W // CHUNK


@functools.partial(
    pl.kernel,
    mesh=plsc.VectorSubcoreMesh(core_axis_name="c", subcore_axis_name="s"),
    out_type=jax.ShapeDtypeStruct((N_ROWS, EMBED_SIZE), jnp.float32),
    scratch_types=[
        pltpu.VMEM((NCHUNKS, CHUNK), jnp.int32),
        pltpu.VMEM((NBUF, CHUNK, EMBED_SIZE), jnp.float32),
    ]
    + [pltpu.SemaphoreType.DMA] * (2 * NBUF),
)
def _gather_rows(idx_hbm, table_hbm, out_hbm, idx_v, rows_v, *sems):
    gsem, osem = sems[:NBUF], sems[NBUF:]
    wid = lax.axis_index("s") * NUM_CORES + lax.axis_index("c")
    base = wid * ROWS_PER_W
    pltpu.sync_copy(idx_hbm.at[wid], idx_v)

    def start_gather(j, b):
        pltpu.async_copy(table_hbm.at[idx_v.at[j]], rows_v.at[b], gsem[b])

    def start_out(j, b):
        pltpu.async_copy(rows_v.at[b],
                         out_hbm.at[pl.ds(base + j * CHUNK, CHUNK)], osem[b])

    def wait(sem):
        pltpu.make_async_copy(
            rows_v.at[0], out_hbm.at[pl.ds(base, CHUNK)], sem).wait()

    for b in range(NBUF):
        start_gather(b, b)
    for b in range(NBUF):
        wait(gsem[b])
        start_out(b, b)

    def body(i, carry):
        for b in range(NBUF):
            wait(osem[b])
            start_gather(i * NBUF + b, b)
        for b in range(NBUF):
            wait(gsem[b])
            start_out(i * NBUF + b, b)
        return carry

    lax.fori_loop(1, NCHUNKS // NBUF, body, 0)
    for b in range(NBUF):
        wait(osem[b])


def kernel(x, PE_weight):
    idx = x.reshape(NW, NCHUNKS, CHUNK).astype(jnp.int32)
    out = _gather_rows(idx, PE_weight)
    return out.reshape(BATCH, SEQ, EMBED_SIZE)
